# Initial kernel scaffold; baseline (speedup 1.0000x reference)
import jax
import jax.numpy as jnp
from jax.experimental import pallas as pl


def kernel(x, edge_index, edge_attr, u, batch, We0, be0, We1, be1, We2, be2, Wn0, bn0, Wn1, bn1, Wn2, bn2, Wg0, bg0, Wg1, bg1, Wg2, bg2):
    raise NotImplementedError("write your pallas kernel here")


# speedup vs baseline: 6.9008x; 6.9008x over previous
import functools

import jax
import jax.numpy as jnp
from jax import lax
from jax.experimental import pallas as pl
from jax.experimental.pallas import tpu as pltpu
from jax.experimental.pallas import tpu_sc as plsc

_NC = 2
_NS = 16


def _relu(v):
    return jnp.maximum(v, 0.0)


def _dot(a, b):
    return jnp.dot(a, b, preferred_element_type=jnp.float32)


def _seg_dot(a, b):
    return lax.dot_general(a, b, (((0,), (0,)), ((), ())),
                           preferred_element_type=jnp.float32)


def kernel(x, edge_index, edge_attr, u, batch,
           We0, be0, We1, be1, We2, be2,
           Wn0, bn0, Wn1, bn1, Wn2, bn2,
           Wg0, bg0, Wg1, bg1, Wg2, bg2):
    N, D = x.shape
    E = edge_attr.shape[0]
    B = u.shape[0]
    row, col = edge_index[0], edge_index[1]

    Wa, Wb, Wc, Wd = We0[:D], We0[D:2 * D], We0[2 * D:3 * D], We0[3 * D:]
    Wn0a, Wn0b, Wn0u = Wn0[:D], Wn0[D:2 * D], Wn0[2 * D:]
    Wg0e, Wg0v, Wg0u = Wg0[:D], Wg0[D:2 * D], Wg0[2 * D:]

    NB = 2000
    nb = N // NB
    batch3 = batch.reshape(nb, 1, NB)

    def onehot_t(b_row):
        return (b_row == lax.broadcasted_iota(jnp.int32, (B, NB), 0)
                ).astype(jnp.float32)

    def prep_body(x_ref, b_ref, u_ref, wa_ref, wb_ref, wd_ref, be0_ref,
                  xa_ref, xb_ref):
        xblk = x_ref[...]
        oh_t = onehot_t(b_ref[0])
        up = _dot(u_ref[...], wd_ref[...])
        xa_ref[...] = (_dot(xblk, wa_ref[...]) + _seg_dot(oh_t, up)
                       + be0_ref[...])
        xb_ref[...] = _dot(xblk, wb_ref[...])

    xa, xb = pl.pallas_call(
        prep_body,
        grid=(nb,),
        in_specs=[
            pl.BlockSpec((NB, D), lambda i: (i, 0)),
            pl.BlockSpec((1, 1, NB), lambda i: (i, 0, 0)),
            pl.BlockSpec((B, D), lambda i: (0, 0)),
            pl.BlockSpec((D, D), lambda i: (0, 0)),
            pl.BlockSpec((D, D), lambda i: (0, 0)),
            pl.BlockSpec((D, D), lambda i: (0, 0)),
            pl.BlockSpec((1, D), lambda i: (0, 0)),
        ],
        out_specs=[pl.BlockSpec((NB, D), lambda i: (i, 0)),
                   pl.BlockSpec((NB, D), lambda i: (i, 0))],
        out_shape=[jax.ShapeDtypeStruct((N, D), jnp.float32),
                   jax.ShapeDtypeStruct((N, D), jnp.float32)],
    )(x, batch3, u, Wa, Wb, Wd, be0.reshape(1, D))

    mesh = plsc.VectorSubcoreMesh(core_axis_name="c", subcore_axis_name="s",
                                  num_cores=_NC, num_subcores=_NS)
    CH = 400
    per_w = E // (_NC * _NS)

    @functools.partial(
        pl.kernel,
        out_type=jax.ShapeDtypeStruct((E, D), jnp.float32),
        mesh=mesh,
        scratch_types=[
            pltpu.VMEM((CH,), jnp.int32),
            pltpu.VMEM((CH,), jnp.int32),
            pltpu.VMEM((CH, D), jnp.float32),
            pltpu.VMEM((CH, D), jnp.float32),
            pltpu.SemaphoreType.DMA,
            pltpu.SemaphoreType.DMA,
        ],
    )
    def gather_k(xa_h, xb_h, row_h, col_h, out_h,
                 idxr, idxc, bufa, bufb, sema, semb):
        wid = lax.axis_index("s") * _NC + lax.axis_index("c")
        base = wid * per_w

        def chunk(i, carry):
            off = base + i * CH
            pltpu.sync_copy(row_h.at[pl.ds(off, CH)], idxr)
            pltpu.sync_copy(col_h.at[pl.ds(off, CH)], idxc)
            ca = pltpu.async_copy(xa_h.at[idxr], bufa, sema)
            cb = pltpu.async_copy(xb_h.at[idxc], bufb, semb)
            ca.wait()
            cb.wait()

            def addrow(j, c2):
                for k in range(D // 16):
                    sl = pl.ds(k * 16, 16)
                    bufa[j, sl] = bufa[j, sl] + bufb[j, sl]
                return c2

            lax.fori_loop(0, CH, addrow, 0)
            pltpu.sync_copy(bufa, out_h.at[pl.ds(off, CH)])
            return carry

        lax.fori_loop(0, per_w // CH, chunk, 0)

    g = gather_k(xa, xb, row, col)

    EB = 2000
    ne = E // EB

    def edge_body(g_ref, ea_ref, wc_ref, w1_ref, w2_ref, b1_ref, b2_ref,
                  out_ref):
        h = _relu(g_ref[...] + _dot(ea_ref[...], wc_ref[...]))
        h = _relu(_dot(h, w1_ref[...]) + b1_ref[...])
        out_ref[...] = _relu(_dot(h, w2_ref[...]) + b2_ref[...])

    e_out = pl.pallas_call(
        edge_body,
        grid=(ne,),
        in_specs=[
            pl.BlockSpec((EB, D), lambda i: (i, 0)),
            pl.BlockSpec((EB, D), lambda i: (i, 0)),
            pl.BlockSpec((D, D), lambda i: (0, 0)),
            pl.BlockSpec((D, D), lambda i: (0, 0)),
            pl.BlockSpec((D, D), lambda i: (0, 0)),
            pl.BlockSpec((1, D), lambda i: (0, 0)),
            pl.BlockSpec((1, D), lambda i: (0, 0)),
        ],
        out_specs=pl.BlockSpec((EB, D), lambda i: (i, 0)),
        out_shape=jax.ShapeDtypeStruct((E, D), jnp.float32),
    )(g, edge_attr, Wc, We1, We2, be1.reshape(1, D), be2.reshape(1, D))

    HN = N // _NC
    SPREAD = 512
    DBASE = (HN + 7) // 8 * 8
    per_tile = E // _NS
    WBS = (HN // _NS) // 8 * 8
    TS = HN - _NS * WBS
    WBC = (N // _NS) // 8 * 8
    TLC = N - _NS * WBC
    ZR = 104

    def scatter_body(eout_h, row_h, sums_h, cnts_h,
                     idx, upd, ones1, zb2, zb1, sh_c, sh_s):
        cid = lax.axis_index("c")
        sid = lax.axis_index("s")

        def zrow(j, c):
            for k in range(D // 16):
                zb2[j, pl.ds(16 * k, 16)] = jnp.zeros((16,), jnp.float32)
            return c

        lax.fori_loop(0, ZR, zrow, 0)

        def fill1(buf, val, n16):
            def body(j, c):
                buf[pl.ds(j * 16, 16)] = jnp.full((16,), val, jnp.float32)
                return c
            lax.fori_loop(0, n16, body, 0)

        fill1(zb1, 0.0, WBC // 16)
        fill1(ones1, 1.0, CH // 16)

        nfull = WBS // ZR
        rem = WBS - nfull * ZR

        def zinit(r, c):
            pltpu.sync_copy(zb2, sh_s.at[pl.ds(sid * WBS + r * ZR, ZR)])
            return c

        lax.fori_loop(0, nfull, zinit, 0)
        if rem:
            pltpu.sync_copy(zb2.at[pl.ds(0, rem)],
                            sh_s.at[pl.ds(sid * WBS + nfull * ZR, rem)])
        pltpu.sync_copy(zb1, sh_c.at[pl.ds(sid * WBC, WBC)])

        @pl.when(sid == _NS - 1)
        def _():
            if TS:
                pltpu.sync_copy(zb2.at[pl.ds(0, TS)],
                                sh_s.at[pl.ds(_NS * WBS, TS)])
            pltpu.sync_copy(zb1.at[pl.ds(0, TLC)],
                            sh_c.at[pl.ds(_NS * WBC, TLC)])

        plsc.subcore_barrier()

        base = sid * per_tile
        lo = cid * HN

        def chunk(i, c):
            off = base + i * CH
            pltpu.sync_copy(row_h.at[pl.ds(off, CH)], idx)
            pltpu.sync_copy(eout_h.at[pl.ds(off, CH)], upd)

            @pl.when(cid == 0)
            def _():
                pltpu.sync_copy(ones1, sh_c.at[idx], add=True)

            def remap(j, c2):
                iv = idx[pl.ds(j * 16, 16)]
                shf = iv - lo
                inb = (shf >= 0) & (shf < HN)
                idx[pl.ds(j * 16, 16)] = jnp.where(
                    inb, shf, DBASE + (iv & (SPREAD - 1)))
                return c2

            lax.fori_loop(0, CH // 16, remap, 0)
            pltpu.sync_copy(upd, sh_s.at[idx], add=True)
            return c

        lax.fori_loop(0, per_tile // CH, chunk, 0)
        plsc.subcore_barrier()

        def wback(r, c):
            rows = pl.ds(sid * WBS + r * ZR, ZR)
            pltpu.sync_copy(sh_s.at[rows], zb2)
            pltpu.sync_copy(
                zb2, sums_h.at[pl.ds(lo + sid * WBS + r * ZR, ZR)])
            return c

        lax.fori_loop(0, nfull, wback, 0)
        if rem:
            pltpu.sync_copy(sh_s.at[pl.ds(sid * WBS + nfull * ZR, rem)],
                            zb2.at[pl.ds(0, rem)])
            pltpu.sync_copy(
                zb2.at[pl.ds(0, rem)],
                sums_h.at[pl.ds(lo + sid * WBS + nfull * ZR, rem)])
        pltpu.sync_copy(sh_c.at[pl.ds(sid * WBC, WBC)], zb1)
        pltpu.sync_copy(zb1, cnts_h.at[pl.ds(cid * N + sid * WBC, WBC)])

        @pl.when(sid == _NS - 1)
        def _():
            if TS:
                tr = pl.ds(_NS * WBS, TS)
                pltpu.sync_copy(sh_s.at[tr], zb2.at[pl.ds(0, TS)])
                pltpu.sync_copy(zb2.at[pl.ds(0, TS)],
                                sums_h.at[pl.ds(lo + _NS * WBS, TS)])
            trc = pl.ds(_NS * WBC, TLC)
            pltpu.sync_copy(sh_c.at[trc], zb1.at[pl.ds(0, TLC)])
            pltpu.sync_copy(zb1.at[pl.ds(0, TLC)],
                            cnts_h.at[pl.ds(cid * N + _NS * WBC, TLC)])

    scatter_k = pl.kernel(
        scatter_body,
        out_type=(jax.ShapeDtypeStruct((N, D), jnp.float32),
                  jax.ShapeDtypeStruct((_NC * N,), jnp.float32)),
        mesh=mesh,
        scratch_types=[
            pltpu.VMEM((CH,), jnp.int32),
            pltpu.VMEM((CH, D), jnp.float32),
            pltpu.VMEM((CH,), jnp.float32),
            pltpu.VMEM((ZR, D), jnp.float32),
            pltpu.VMEM((WBC,), jnp.float32),
            pltpu.VMEM_SHARED((N,), jnp.float32),
            pltpu.VMEM_SHARED((DBASE + SPREAD, D), jnp.float32),
        ],
    )

    sums, cnts = scatter_k(e_out, row)

    ecnt = jnp.broadcast_to(
        jnp.maximum(cnts[:N] + cnts[N:], 1.0)[:, None], (N, D))

    def node_body(x_ref, s_ref, ec_ref, b_ref, u_ref,
                  wn0a_ref, wn0b_ref, wn0u_ref, wn1_ref, wn2_ref,
                  bn0_ref, bn1_ref, bn2_ref,
                  wg0e_ref, wg0v_ref, wg0u_ref, wg1_ref, wg2_ref,
                  bg0_ref, bg1_ref, bg2_ref,
                  xo_ref, uo_ref, es_ref, vs_ref, nc_ref):
        i = pl.program_id(0)
        v_e = s_ref[...] / ec_ref[...]
        oh_t = onehot_t(b_ref[0])
        up = _dot(u_ref[...], wn0u_ref[...])
        h = _relu(_dot(x_ref[...], wn0a_ref[...]) + _dot(v_e, wn0b_ref[...])
                  + _seg_dot(oh_t, up) + bn0_ref[...])
        h = _relu(_dot(h, wn1_ref[...]) + bn1_ref[...])
        xo = _relu(_dot(h, wn2_ref[...]) + bn2_ref[...])
        xo_ref[...] = xo

        @pl.when(i == 0)
        def _():
            es_ref[...] = jnp.zeros((B, D), jnp.float32)
            vs_ref[...] = jnp.zeros((B, D), jnp.float32)
            nc_ref[...] = jnp.zeros((B, D), jnp.float32)

        es_ref[...] = es_ref[...] + _dot(oh_t, v_e)
        vs_ref[...] = vs_ref[...] + _dot(oh_t, xo)
        nc_ref[...] = nc_ref[...] + _dot(oh_t, jnp.ones((NB, D), jnp.float32))

        @pl.when(i == nb - 1)
        def _():
            ncv = jnp.maximum(nc_ref[...], 1.0)
            u_e = es_ref[...] / ncv
            u_v = vs_ref[...] / ncv
            gh = _relu(_dot(u_e, wg0e_ref[...]) + _dot(u_v, wg0v_ref[...])
                       + _dot(u_ref[...], wg0u_ref[...]) + bg0_ref[...])
            gh = _relu(_dot(gh, wg1_ref[...]) + bg1_ref[...])
            uo_ref[...] = _relu(_dot(gh, wg2_ref[...]) + bg2_ref[...])

    wspec = pl.BlockSpec((D, D), lambda i: (0, 0))
    bspec = pl.BlockSpec((1, D), lambda i: (0, 0))
    x_out, u_out = pl.pallas_call(
        node_body,
        grid=(nb,),
        in_specs=[
            pl.BlockSpec((NB, D), lambda i: (i, 0)),
            pl.BlockSpec((NB, D), lambda i: (i, 0)),
            pl.BlockSpec((NB, D), lambda i: (i, 0)),
            pl.BlockSpec((1, 1, NB), lambda i: (i, 0, 0)),
            pl.BlockSpec((B, D), lambda i: (0, 0)),
            wspec, wspec, wspec, wspec, wspec, bspec, bspec, bspec,
            wspec, wspec, wspec, wspec, wspec, bspec, bspec, bspec,
        ],
        out_specs=[pl.BlockSpec((NB, D), lambda i: (i, 0)),
                   pl.BlockSpec((B, D), lambda i: (0, 0))],
        out_shape=[jax.ShapeDtypeStruct((N, D), jnp.float32),
                   jax.ShapeDtypeStruct((B, D), jnp.float32)],
        scratch_shapes=[pltpu.VMEM((B, D), jnp.float32),
                        pltpu.VMEM((B, D), jnp.float32),
                        pltpu.VMEM((B, D), jnp.float32)],
    )(x, sums, ecnt, batch3, u,
      Wn0a, Wn0b, Wn0u, Wn1, Wn2,
      bn0.reshape(1, D), bn1.reshape(1, D), bn2.reshape(1, D),
      Wg0e, Wg0v, Wg0u, Wg1, Wg2,
      bg0.reshape(1, D), bg1.reshape(1, D), bg2.reshape(1, D))

    return (x_out, e_out, u_out)


# speedup vs baseline: 8.5348x; 1.2368x over previous
import functools

import jax
import jax.numpy as jnp
from jax import lax
from jax.experimental import pallas as pl
from jax.experimental.pallas import tpu as pltpu
from jax.experimental.pallas import tpu_sc as plsc

_NC = 2
_NS = 16


def _relu(v):
    return jnp.maximum(v, 0.0)


def _dot(a, b):
    return jnp.dot(a, b, preferred_element_type=jnp.float32)


def _seg_dot(a, b):
    return lax.dot_general(a, b, (((0,), (0,)), ((), ())),
                           preferred_element_type=jnp.float32)


def kernel(x, edge_index, edge_attr, u, batch,
           We0, be0, We1, be1, We2, be2,
           Wn0, bn0, Wn1, bn1, Wn2, bn2,
           Wg0, bg0, Wg1, bg1, Wg2, bg2):
    N, D = x.shape
    E = edge_attr.shape[0]
    B = u.shape[0]
    row, col = edge_index[0], edge_index[1]

    Wa, Wb, Wc, Wd = We0[:D], We0[D:2 * D], We0[2 * D:3 * D], We0[3 * D:]
    Wn0a, Wn0b, Wn0u = Wn0[:D], Wn0[D:2 * D], Wn0[2 * D:]
    Wg0e, Wg0v, Wg0u = Wg0[:D], Wg0[D:2 * D], Wg0[2 * D:]

    NB = 2000
    nb = N // NB
    batch3 = batch.reshape(nb, 1, NB)

    def onehot_t(b_row):
        return (b_row == lax.broadcasted_iota(jnp.int32, (B, NB), 0)
                ).astype(jnp.float32)

    def prep_body(x_ref, b_ref, u_ref, wa_ref, wb_ref, wd_ref, be0_ref,
                  xa_ref, xb_ref):
        xblk = x_ref[...]
        oh_t = onehot_t(b_ref[0])
        up = _dot(u_ref[...], wd_ref[...])
        xa_ref[...] = (_dot(xblk, wa_ref[...]) + _seg_dot(oh_t, up)
                       + be0_ref[...])
        xb_ref[...] = _dot(xblk, wb_ref[...])

    xa, xb = pl.pallas_call(
        prep_body,
        grid=(nb,),
        in_specs=[
            pl.BlockSpec((NB, D), lambda i: (i, 0)),
            pl.BlockSpec((1, 1, NB), lambda i: (i, 0, 0)),
            pl.BlockSpec((B, D), lambda i: (0, 0)),
            pl.BlockSpec((D, D), lambda i: (0, 0)),
            pl.BlockSpec((D, D), lambda i: (0, 0)),
            pl.BlockSpec((D, D), lambda i: (0, 0)),
            pl.BlockSpec((1, D), lambda i: (0, 0)),
        ],
        out_specs=[pl.BlockSpec((NB, D), lambda i: (i, 0)),
                   pl.BlockSpec((NB, D), lambda i: (i, 0))],
        out_shape=[jax.ShapeDtypeStruct((N, D), jnp.float32),
                   jax.ShapeDtypeStruct((N, D), jnp.float32)],
    )(x, batch3, u, Wa, Wb, Wd, be0.reshape(1, D))

    mesh = plsc.VectorSubcoreMesh(core_axis_name="c", subcore_axis_name="s",
                                  num_cores=_NC, num_subcores=_NS)
    CH = 200
    per_w = E // (_NC * _NS)
    NCHG = per_w // CH

    @functools.partial(
        pl.kernel,
        out_type=jax.ShapeDtypeStruct((E, D), jnp.float32),
        mesh=mesh,
        scratch_types=[
            [pltpu.VMEM((CH,), jnp.int32)] * 2,
            [pltpu.VMEM((CH,), jnp.int32)] * 2,
            [pltpu.VMEM((CH, D), jnp.float32)] * 2,
            [pltpu.VMEM((CH, D), jnp.float32)] * 2,
            [pltpu.SemaphoreType.DMA] * 2,
            [pltpu.SemaphoreType.DMA] * 2,
        ],
    )
    def gather_k(xa_h, xb_h, row_h, col_h, out_h,
                 idxr, idxc, bufa, bufb, sema, semb):
        wid = lax.axis_index("s") * _NC + lax.axis_index("c")
        base = wid * per_w

        def start(i, p):
            off = base + i * CH
            pltpu.sync_copy(row_h.at[pl.ds(off, CH)], idxr[p])
            pltpu.sync_copy(col_h.at[pl.ds(off, CH)], idxc[p])
            pltpu.async_copy(xa_h.at[idxr[p]], bufa[p], sema[p])
            pltpu.async_copy(xb_h.at[idxc[p]], bufb[p], semb[p])

        def finish(i, p):
            pltpu.make_async_copy(xa_h.at[idxr[p]], bufa[p], sema[p]).wait()
            pltpu.make_async_copy(xb_h.at[idxc[p]], bufb[p], semb[p]).wait()

            def addrow(j, c2):
                for k in range(D // 16):
                    sl = pl.ds(k * 16, 16)
                    bufa[p][j, sl] = bufa[p][j, sl] + bufb[p][j, sl]
                return c2

            lax.fori_loop(0, CH, addrow, 0)
            pltpu.sync_copy(bufa[p], out_h.at[pl.ds(base + i * CH, CH)])

        start(0, 0)

        def chunk(i, c):
            @pl.when(lax.rem(i, 2) == 0)
            def _():
                @pl.when(i + 1 < NCHG)
                def _():
                    start(i + 1, 1)
                finish(i, 0)

            @pl.when(lax.rem(i, 2) == 1)
            def _():
                @pl.when(i + 1 < NCHG)
                def _():
                    start(i + 1, 0)
                finish(i, 1)

            return c

        lax.fori_loop(0, NCHG, chunk, 0)

    g = gather_k(xa, xb, row, col)

    EB = 2000
    ne = E // EB

    def edge_body(g_ref, ea_ref, wc_ref, w1_ref, w2_ref, b1_ref, b2_ref,
                  out_ref):
        h = _relu(g_ref[...] + _dot(ea_ref[...], wc_ref[...]))
        h = _relu(_dot(h, w1_ref[...]) + b1_ref[...])
        out_ref[...] = _relu(_dot(h, w2_ref[...]) + b2_ref[...])

    e_out = pl.pallas_call(
        edge_body,
        grid=(ne,),
        in_specs=[
            pl.BlockSpec((EB, D), lambda i: (i, 0)),
            pl.BlockSpec((EB, D), lambda i: (i, 0)),
            pl.BlockSpec((D, D), lambda i: (0, 0)),
            pl.BlockSpec((D, D), lambda i: (0, 0)),
            pl.BlockSpec((D, D), lambda i: (0, 0)),
            pl.BlockSpec((1, D), lambda i: (0, 0)),
            pl.BlockSpec((1, D), lambda i: (0, 0)),
        ],
        out_specs=pl.BlockSpec((EB, D), lambda i: (i, 0)),
        out_shape=jax.ShapeDtypeStruct((E, D), jnp.float32),
    )(g, edge_attr, Wc, We1, We2, be1.reshape(1, D), be2.reshape(1, D))

    HN = N // _NC
    SPREAD = 512
    DBASE = (HN + 7) // 8 * 8
    per_tile = E // _NS
    WBS = (HN // _NS) // 8 * 8
    TS = HN - _NS * WBS
    WBC = (N // _NS) // 8 * 8
    TLC = N - _NS * WBC
    ZR = 104

    CHS = 160
    NCHS = per_tile // CHS

    def scatter_body(eout_h, row_h, sums_h, cnts_h,
                     idx, idx2, upd, ones1, zb2, zb1,
                     sem_l, sh_c, sh_s):
        cid = lax.axis_index("c")
        sid = lax.axis_index("s")

        def zrow(j, c):
            for k in range(D // 16):
                zb2[j, pl.ds(16 * k, 16)] = jnp.zeros((16,), jnp.float32)
            return c

        lax.fori_loop(0, ZR, zrow, 0)

        def fill1(buf, val, n16):
            def body(j, c):
                buf[pl.ds(j * 16, 16)] = jnp.full((16,), val, jnp.float32)
                return c
            lax.fori_loop(0, n16, body, 0)

        fill1(zb1, 0.0, WBC // 16)
        fill1(ones1, 1.0, CHS // 16)

        nfull = WBS // ZR
        rem = WBS - nfull * ZR

        def zinit(r, c):
            pltpu.sync_copy(zb2, sh_s.at[pl.ds(sid * WBS + r * ZR, ZR)])
            return c

        lax.fori_loop(0, nfull, zinit, 0)
        if rem:
            pltpu.sync_copy(zb2.at[pl.ds(0, rem)],
                            sh_s.at[pl.ds(sid * WBS + nfull * ZR, rem)])
        pltpu.sync_copy(zb1, sh_c.at[pl.ds(sid * WBC, WBC)])

        @pl.when(sid == _NS - 1)
        def _():
            if TS:
                pltpu.sync_copy(zb2.at[pl.ds(0, TS)],
                                sh_s.at[pl.ds(_NS * WBS, TS)])
            pltpu.sync_copy(zb1.at[pl.ds(0, TLC)],
                            sh_c.at[pl.ds(_NS * WBC, TLC)])

        plsc.subcore_barrier()

        base = sid * per_tile
        lo = cid * HN

        def load(i, p):
            off = base + i * CHS
            pltpu.sync_copy(row_h.at[pl.ds(off, CHS)], idx[p])
            pltpu.async_copy(eout_h.at[pl.ds(off, CHS)], upd[p], sem_l[p])

        def process(i, p):
            pltpu.make_async_copy(eout_h.at[pl.ds(base + i * CHS, CHS)],
                                  upd[p], sem_l[p]).wait()

            def remap(j, c2):
                iv = idx[p][pl.ds(j * 16, 16)]
                shf = iv - lo
                inb = (shf >= 0) & (shf < HN)
                idx2[p][pl.ds(j * 16, 16)] = jnp.where(
                    inb, shf, DBASE + (iv & (SPREAD - 1)))
                return c2

            lax.fori_loop(0, CHS // 16, remap, 0)

            @pl.when(cid == 0)
            def _():
                pltpu.sync_copy(ones1, sh_c.at[idx[p]], add=True)

            pltpu.sync_copy(upd[p], sh_s.at[idx2[p]], add=True)

        load(0, 0)

        def chunk(i, c):
            @pl.when(lax.rem(i, 2) == 0)
            def _():
                @pl.when(i + 1 < NCHS)
                def _():
                    load(i + 1, 1)
                process(i, 0)

            @pl.when(lax.rem(i, 2) == 1)
            def _():
                @pl.when(i + 1 < NCHS)
                def _():
                    load(i + 1, 0)
                process(i, 1)

            return c

        lax.fori_loop(0, NCHS, chunk, 0)
        plsc.subcore_barrier()

        def wback(r, c):
            rows = pl.ds(sid * WBS + r * ZR, ZR)
            pltpu.sync_copy(sh_s.at[rows], zb2)
            pltpu.sync_copy(
                zb2, sums_h.at[pl.ds(lo + sid * WBS + r * ZR, ZR)])
            return c

        lax.fori_loop(0, nfull, wback, 0)
        if rem:
            pltpu.sync_copy(sh_s.at[pl.ds(sid * WBS + nfull * ZR, rem)],
                            zb2.at[pl.ds(0, rem)])
            pltpu.sync_copy(
                zb2.at[pl.ds(0, rem)],
                sums_h.at[pl.ds(lo + sid * WBS + nfull * ZR, rem)])
        pltpu.sync_copy(sh_c.at[pl.ds(sid * WBC, WBC)], zb1)
        pltpu.sync_copy(zb1, cnts_h.at[pl.ds(cid * N + sid * WBC, WBC)])

        @pl.when(sid == _NS - 1)
        def _():
            if TS:
                tr = pl.ds(_NS * WBS, TS)
                pltpu.sync_copy(sh_s.at[tr], zb2.at[pl.ds(0, TS)])
                pltpu.sync_copy(zb2.at[pl.ds(0, TS)],
                                sums_h.at[pl.ds(lo + _NS * WBS, TS)])
            trc = pl.ds(_NS * WBC, TLC)
            pltpu.sync_copy(sh_c.at[trc], zb1.at[pl.ds(0, TLC)])
            pltpu.sync_copy(zb1.at[pl.ds(0, TLC)],
                            cnts_h.at[pl.ds(cid * N + _NS * WBC, TLC)])

    scatter_k = pl.kernel(
        scatter_body,
        out_type=(jax.ShapeDtypeStruct((N, D), jnp.float32),
                  jax.ShapeDtypeStruct((_NC * N,), jnp.float32)),
        mesh=mesh,
        scratch_types=[
            [pltpu.VMEM((CHS,), jnp.int32)] * 2,
            [pltpu.VMEM((CHS,), jnp.int32)] * 2,
            [pltpu.VMEM((CHS, D), jnp.float32)] * 2,
            pltpu.VMEM((CHS,), jnp.float32),
            pltpu.VMEM((ZR, D), jnp.float32),
            pltpu.VMEM((WBC,), jnp.float32),
            [pltpu.SemaphoreType.DMA] * 2,
            pltpu.VMEM_SHARED((N,), jnp.float32),
            pltpu.VMEM_SHARED((DBASE + SPREAD, D), jnp.float32),
        ],
    )

    sums, cnts = scatter_k(e_out, row)

    ecnt = jnp.broadcast_to(
        jnp.maximum(cnts[:N] + cnts[N:], 1.0)[:, None], (N, D))

    def node_body(x_ref, s_ref, ec_ref, b_ref, u_ref,
                  wn0a_ref, wn0b_ref, wn0u_ref, wn1_ref, wn2_ref,
                  bn0_ref, bn1_ref, bn2_ref,
                  wg0e_ref, wg0v_ref, wg0u_ref, wg1_ref, wg2_ref,
                  bg0_ref, bg1_ref, bg2_ref,
                  xo_ref, uo_ref, es_ref, vs_ref, nc_ref):
        i = pl.program_id(0)
        v_e = s_ref[...] / ec_ref[...]
        oh_t = onehot_t(b_ref[0])
        up = _dot(u_ref[...], wn0u_ref[...])
        h = _relu(_dot(x_ref[...], wn0a_ref[...]) + _dot(v_e, wn0b_ref[...])
                  + _seg_dot(oh_t, up) + bn0_ref[...])
        h = _relu(_dot(h, wn1_ref[...]) + bn1_ref[...])
        xo = _relu(_dot(h, wn2_ref[...]) + bn2_ref[...])
        xo_ref[...] = xo

        @pl.when(i == 0)
        def _():
            es_ref[...] = jnp.zeros((B, D), jnp.float32)
            vs_ref[...] = jnp.zeros((B, D), jnp.float32)
            nc_ref[...] = jnp.zeros((B, D), jnp.float32)

        es_ref[...] = es_ref[...] + _dot(oh_t, v_e)
        vs_ref[...] = vs_ref[...] + _dot(oh_t, xo)
        nc_ref[...] = nc_ref[...] + _dot(oh_t, jnp.ones((NB, D), jnp.float32))

        @pl.when(i == nb - 1)
        def _():
            ncv = jnp.maximum(nc_ref[...], 1.0)
            u_e = es_ref[...] / ncv
            u_v = vs_ref[...] / ncv
            gh = _relu(_dot(u_e, wg0e_ref[...]) + _dot(u_v, wg0v_ref[...])
                       + _dot(u_ref[...], wg0u_ref[...]) + bg0_ref[...])
            gh = _relu(_dot(gh, wg1_ref[...]) + bg1_ref[...])
            uo_ref[...] = _relu(_dot(gh, wg2_ref[...]) + bg2_ref[...])

    wspec = pl.BlockSpec((D, D), lambda i: (0, 0))
    bspec = pl.BlockSpec((1, D), lambda i: (0, 0))
    x_out, u_out = pl.pallas_call(
        node_body,
        grid=(nb,),
        in_specs=[
            pl.BlockSpec((NB, D), lambda i: (i, 0)),
            pl.BlockSpec((NB, D), lambda i: (i, 0)),
            pl.BlockSpec((NB, D), lambda i: (i, 0)),
            pl.BlockSpec((1, 1, NB), lambda i: (i, 0, 0)),
            pl.BlockSpec((B, D), lambda i: (0, 0)),
            wspec, wspec, wspec, wspec, wspec, bspec, bspec, bspec,
            wspec, wspec, wspec, wspec, wspec, bspec, bspec, bspec,
        ],
        out_specs=[pl.BlockSpec((NB, D), lambda i: (i, 0)),
                   pl.BlockSpec((B, D), lambda i: (0, 0))],
        out_shape=[jax.ShapeDtypeStruct((N, D), jnp.float32),
                   jax.ShapeDtypeStruct((B, D), jnp.float32)],
        scratch_shapes=[pltpu.VMEM((B, D), jnp.float32),
                        pltpu.VMEM((B, D), jnp.float32),
                        pltpu.VMEM((B, D), jnp.float32)],
    )(x, sums, ecnt, batch3, u,
      Wn0a, Wn0b, Wn0u, Wn1, Wn2,
      bn0.reshape(1, D), bn1.reshape(1, D), bn2.reshape(1, D),
      Wg0e, Wg0v, Wg0u, Wg1, Wg2,
      bg0.reshape(1, D), bg1.reshape(1, D), bg2.reshape(1, D))

    return (x_out, e_out, u_out)


# speedup vs baseline: 8.5908x; 1.0066x over previous
import functools

import jax
import jax.numpy as jnp
from jax import lax
from jax.experimental import pallas as pl
from jax.experimental.pallas import tpu as pltpu
from jax.experimental.pallas import tpu_sc as plsc

_NC = 2
_NS = 16


def _relu(v):
    return jnp.maximum(v, 0.0)


def _dot(a, b):
    return jnp.dot(a, b, preferred_element_type=jnp.float32)


def _seg_dot(a, b):
    return lax.dot_general(a, b, (((0,), (0,)), ((), ())),
                           preferred_element_type=jnp.float32)


def kernel(x, edge_index, edge_attr, u, batch,
           We0, be0, We1, be1, We2, be2,
           Wn0, bn0, Wn1, bn1, Wn2, bn2,
           Wg0, bg0, Wg1, bg1, Wg2, bg2):
    N, D = x.shape
    E = edge_attr.shape[0]
    B = u.shape[0]
    row, col = edge_index[0], edge_index[1]

    Wa, Wb, Wc, Wd = We0[:D], We0[D:2 * D], We0[2 * D:3 * D], We0[3 * D:]
    Wn0a, Wn0b, Wn0u = Wn0[:D], Wn0[D:2 * D], Wn0[2 * D:]
    Wg0e, Wg0v, Wg0u = Wg0[:D], Wg0[D:2 * D], Wg0[2 * D:]

    NB = 2000
    nb = N // NB
    batch3 = batch.reshape(nb, 1, NB)

    def onehot_t(b_row):
        return (b_row == lax.broadcasted_iota(jnp.int32, (B, NB), 0)
                ).astype(jnp.float32)

    def prep_body(x_ref, b_ref, u_ref, wa_ref, wb_ref, wd_ref, be0_ref,
                  xa_ref, xb_ref):
        xblk = x_ref[...]
        oh_t = onehot_t(b_ref[0])
        up = _dot(u_ref[...], wd_ref[...])
        xa_ref[...] = (_dot(xblk, wa_ref[...]) + _seg_dot(oh_t, up)
                       + be0_ref[...])
        xb_ref[...] = _dot(xblk, wb_ref[...])

    xa, xb = pl.pallas_call(
        prep_body,
        grid=(nb,),
        in_specs=[
            pl.BlockSpec((NB, D), lambda i: (i, 0)),
            pl.BlockSpec((1, 1, NB), lambda i: (i, 0, 0)),
            pl.BlockSpec((B, D), lambda i: (0, 0)),
            pl.BlockSpec((D, D), lambda i: (0, 0)),
            pl.BlockSpec((D, D), lambda i: (0, 0)),
            pl.BlockSpec((D, D), lambda i: (0, 0)),
            pl.BlockSpec((1, D), lambda i: (0, 0)),
        ],
        out_specs=[pl.BlockSpec((NB, D), lambda i: (i, 0)),
                   pl.BlockSpec((NB, D), lambda i: (i, 0))],
        out_shape=[jax.ShapeDtypeStruct((N, D), jnp.float32),
                   jax.ShapeDtypeStruct((N, D), jnp.float32)],
    )(x, batch3, u, Wa, Wb, Wd, be0.reshape(1, D))

    mesh = plsc.VectorSubcoreMesh(core_axis_name="c", subcore_axis_name="s",
                                  num_cores=_NC, num_subcores=_NS)
    CH = 200
    per_w = E // (_NC * _NS)
    NCHG = per_w // CH

    @functools.partial(
        pl.kernel,
        out_type=jax.ShapeDtypeStruct((E, D), jnp.float32),
        mesh=mesh,
        scratch_types=[
            [pltpu.VMEM((CH,), jnp.int32)] * 2,
            [pltpu.VMEM((CH,), jnp.int32)] * 2,
            [pltpu.VMEM((CH, D), jnp.float32)] * 2,
            [pltpu.VMEM((CH, D), jnp.float32)] * 2,
            [pltpu.SemaphoreType.DMA] * 2,
            [pltpu.SemaphoreType.DMA] * 2,
        ],
    )
    def gather_k(xa_h, xb_h, row_h, col_h, out_h,
                 idxr, idxc, bufa, bufb, sema, semb):
        wid = lax.axis_index("s") * _NC + lax.axis_index("c")
        base = wid * per_w

        def start(i, p):
            off = base + i * CH
            pltpu.sync_copy(row_h.at[pl.ds(off, CH)], idxr[p])
            pltpu.sync_copy(col_h.at[pl.ds(off, CH)], idxc[p])
            pltpu.async_copy(xa_h.at[idxr[p]], bufa[p], sema[p])
            pltpu.async_copy(xb_h.at[idxc[p]], bufb[p], semb[p])

        def finish(i, p):
            pltpu.make_async_copy(xa_h.at[idxr[p]], bufa[p], sema[p]).wait()
            pltpu.make_async_copy(xb_h.at[idxc[p]], bufb[p], semb[p]).wait()

            def addrow(j, c2):
                for k in range(D // 16):
                    sl = pl.ds(k * 16, 16)
                    bufa[p][j, sl] = bufa[p][j, sl] + bufb[p][j, sl]
                return c2

            lax.fori_loop(0, CH, addrow, 0)
            pltpu.sync_copy(bufa[p], out_h.at[pl.ds(base + i * CH, CH)])

        start(0, 0)

        def chunk(i, c):
            @pl.when(lax.rem(i, 2) == 0)
            def _():
                @pl.when(i + 1 < NCHG)
                def _():
                    start(i + 1, 1)
                finish(i, 0)

            @pl.when(lax.rem(i, 2) == 1)
            def _():
                @pl.when(i + 1 < NCHG)
                def _():
                    start(i + 1, 0)
                finish(i, 1)

            return c

        lax.fori_loop(0, NCHG, chunk, 0)

    g = gather_k(xa, xb, row, col)

    EB = 2000
    ne = E // EB

    def edge_body(g_ref, ea_ref, wc_ref, w1_ref, w2_ref, b1_ref, b2_ref,
                  out_ref):
        h = _relu(g_ref[...] + _dot(ea_ref[...], wc_ref[...]))
        h = _relu(_dot(h, w1_ref[...]) + b1_ref[...])
        out_ref[...] = _relu(_dot(h, w2_ref[...]) + b2_ref[...])

    e_out = pl.pallas_call(
        edge_body,
        grid=(ne,),
        in_specs=[
            pl.BlockSpec((EB, D), lambda i: (i, 0)),
            pl.BlockSpec((EB, D), lambda i: (i, 0)),
            pl.BlockSpec((D, D), lambda i: (0, 0)),
            pl.BlockSpec((D, D), lambda i: (0, 0)),
            pl.BlockSpec((D, D), lambda i: (0, 0)),
            pl.BlockSpec((1, D), lambda i: (0, 0)),
            pl.BlockSpec((1, D), lambda i: (0, 0)),
        ],
        out_specs=pl.BlockSpec((EB, D), lambda i: (i, 0)),
        out_shape=jax.ShapeDtypeStruct((E, D), jnp.float32),
    )(g, edge_attr, Wc, We1, We2, be1.reshape(1, D), be2.reshape(1, D))

    HN = N // _NC
    SPREAD = 512
    DBASE = (HN + 7) // 8 * 8
    per_tile = E // _NS
    WBS = (HN // _NS) // 8 * 8
    TS = HN - _NS * WBS
    WBC = (N // _NS) // 8 * 8
    TLC = N - _NS * WBC
    ZR = 104

    CHS = 160
    NCHS = per_tile // CHS

    def scatter_body(eout_h, row_h, sums_h, cnts_h,
                     idx, idx2, upd, ones1, zb2, zb1,
                     sem_l, sem_s, sem_c, sh_c, sh_s):
        cid = lax.axis_index("c")
        sid = lax.axis_index("s")

        def zrow(j, c):
            for k in range(D // 16):
                zb2[j, pl.ds(16 * k, 16)] = jnp.zeros((16,), jnp.float32)
            return c

        lax.fori_loop(0, ZR, zrow, 0)

        def fill1(buf, val, n16):
            def body(j, c):
                buf[pl.ds(j * 16, 16)] = jnp.full((16,), val, jnp.float32)
                return c
            lax.fori_loop(0, n16, body, 0)

        fill1(zb1, 0.0, WBC // 16)
        fill1(ones1, 1.0, CHS // 16)

        nfull = WBS // ZR
        rem = WBS - nfull * ZR

        def zinit(r, c):
            pltpu.sync_copy(zb2, sh_s.at[pl.ds(sid * WBS + r * ZR, ZR)])
            return c

        lax.fori_loop(0, nfull, zinit, 0)
        if rem:
            pltpu.sync_copy(zb2.at[pl.ds(0, rem)],
                            sh_s.at[pl.ds(sid * WBS + nfull * ZR, rem)])
        pltpu.sync_copy(zb1, sh_c.at[pl.ds(sid * WBC, WBC)])

        @pl.when(sid == _NS - 1)
        def _():
            if TS:
                pltpu.sync_copy(zb2.at[pl.ds(0, TS)],
                                sh_s.at[pl.ds(_NS * WBS, TS)])
            pltpu.sync_copy(zb1.at[pl.ds(0, TLC)],
                            sh_c.at[pl.ds(_NS * WBC, TLC)])

        plsc.subcore_barrier()

        base = sid * per_tile
        lo = cid * HN

        def load(i, p):
            off = base + i * CHS
            pltpu.sync_copy(row_h.at[pl.ds(off, CHS)], idx[p])
            pltpu.async_copy(eout_h.at[pl.ds(off, CHS)], upd[p], sem_l[p])

        def process(i, p):
            pltpu.make_async_copy(eout_h.at[pl.ds(base + i * CHS, CHS)],
                                  upd[p], sem_l[p]).wait()

            def remap(j, c2):
                iv = idx[p][pl.ds(j * 16, 16)]
                shf = iv - lo
                inb = (shf >= 0) & (shf < HN)
                idx2[p][pl.ds(j * 16, 16)] = jnp.where(
                    inb, shf, DBASE + (iv & (SPREAD - 1)))
                return c2

            lax.fori_loop(0, CHS // 16, remap, 0)

            @pl.when(cid == 0)
            def _():
                pltpu.async_copy(ones1, sh_c.at[idx[p]], sem_c[p], add=True)

            pltpu.async_copy(upd[p], sh_s.at[idx2[p]], sem_s[p], add=True)

        def wait_scatters(p):
            pltpu.make_async_copy(upd[p], sh_s.at[idx2[p]], sem_s[p]).wait()

            @pl.when(cid == 0)
            def _():
                pltpu.make_async_copy(ones1, sh_c.at[idx[p]],
                                      sem_c[p]).wait()

        load(0, 0)

        def chunk(i, c):
            @pl.when(lax.rem(i, 2) == 0)
            def _():
                @pl.when(i + 1 < NCHS)
                def _():
                    @pl.when(i >= 1)
                    def _():
                        wait_scatters(1)
                    load(i + 1, 1)
                process(i, 0)

            @pl.when(lax.rem(i, 2) == 1)
            def _():
                @pl.when(i + 1 < NCHS)
                def _():
                    wait_scatters(0)
                    load(i + 1, 0)
                process(i, 1)

            return c

        lax.fori_loop(0, NCHS, chunk, 0)
        wait_scatters((NCHS - 2) % 2)
        wait_scatters((NCHS - 1) % 2)
        plsc.subcore_barrier()

        def wback(r, c):
            rows = pl.ds(sid * WBS + r * ZR, ZR)
            pltpu.sync_copy(sh_s.at[rows], zb2)
            pltpu.sync_copy(
                zb2, sums_h.at[pl.ds(lo + sid * WBS + r * ZR, ZR)])
            return c

        lax.fori_loop(0, nfull, wback, 0)
        if rem:
            pltpu.sync_copy(sh_s.at[pl.ds(sid * WBS + nfull * ZR, rem)],
                            zb2.at[pl.ds(0, rem)])
            pltpu.sync_copy(
                zb2.at[pl.ds(0, rem)],
                sums_h.at[pl.ds(lo + sid * WBS + nfull * ZR, rem)])
        pltpu.sync_copy(sh_c.at[pl.ds(sid * WBC, WBC)], zb1)
        pltpu.sync_copy(zb1, cnts_h.at[pl.ds(cid * N + sid * WBC, WBC)])

        @pl.when(sid == _NS - 1)
        def _():
            if TS:
                tr = pl.ds(_NS * WBS, TS)
                pltpu.sync_copy(sh_s.at[tr], zb2.at[pl.ds(0, TS)])
                pltpu.sync_copy(zb2.at[pl.ds(0, TS)],
                                sums_h.at[pl.ds(lo + _NS * WBS, TS)])
            trc = pl.ds(_NS * WBC, TLC)
            pltpu.sync_copy(sh_c.at[trc], zb1.at[pl.ds(0, TLC)])
            pltpu.sync_copy(zb1.at[pl.ds(0, TLC)],
                            cnts_h.at[pl.ds(cid * N + _NS * WBC, TLC)])

    scatter_k = pl.kernel(
        scatter_body,
        out_type=(jax.ShapeDtypeStruct((N, D), jnp.float32),
                  jax.ShapeDtypeStruct((_NC * N,), jnp.float32)),
        mesh=mesh,
        scratch_types=[
            [pltpu.VMEM((CHS,), jnp.int32)] * 2,
            [pltpu.VMEM((CHS,), jnp.int32)] * 2,
            [pltpu.VMEM((CHS, D), jnp.float32)] * 2,
            pltpu.VMEM((CHS,), jnp.float32),
            pltpu.VMEM((ZR, D), jnp.float32),
            pltpu.VMEM((WBC,), jnp.float32),
            [pltpu.SemaphoreType.DMA] * 2,
            [pltpu.SemaphoreType.DMA] * 2,
            [pltpu.SemaphoreType.DMA] * 2,
            pltpu.VMEM_SHARED((N,), jnp.float32),
            pltpu.VMEM_SHARED((DBASE + SPREAD, D), jnp.float32),
        ],
    )

    sums, cnts = scatter_k(e_out, row)

    ecnt = jnp.broadcast_to(
        jnp.maximum(cnts[:N] + cnts[N:], 1.0)[:, None], (N, D))

    def node_body(x_ref, s_ref, ec_ref, b_ref, u_ref,
                  wn0a_ref, wn0b_ref, wn0u_ref, wn1_ref, wn2_ref,
                  bn0_ref, bn1_ref, bn2_ref,
                  wg0e_ref, wg0v_ref, wg0u_ref, wg1_ref, wg2_ref,
                  bg0_ref, bg1_ref, bg2_ref,
                  xo_ref, uo_ref, es_ref, vs_ref, nc_ref):
        i = pl.program_id(0)
        v_e = s_ref[...] / ec_ref[...]
        oh_t = onehot_t(b_ref[0])
        up = _dot(u_ref[...], wn0u_ref[...])
        h = _relu(_dot(x_ref[...], wn0a_ref[...]) + _dot(v_e, wn0b_ref[...])
                  + _seg_dot(oh_t, up) + bn0_ref[...])
        h = _relu(_dot(h, wn1_ref[...]) + bn1_ref[...])
        xo = _relu(_dot(h, wn2_ref[...]) + bn2_ref[...])
        xo_ref[...] = xo

        @pl.when(i == 0)
        def _():
            es_ref[...] = jnp.zeros((B, D), jnp.float32)
            vs_ref[...] = jnp.zeros((B, D), jnp.float32)
            nc_ref[...] = jnp.zeros((B, D), jnp.float32)

        es_ref[...] = es_ref[...] + _dot(oh_t, v_e)
        vs_ref[...] = vs_ref[...] + _dot(oh_t, xo)
        nc_ref[...] = nc_ref[...] + _dot(oh_t, jnp.ones((NB, D), jnp.float32))

        @pl.when(i == nb - 1)
        def _():
            ncv = jnp.maximum(nc_ref[...], 1.0)
            u_e = es_ref[...] / ncv
            u_v = vs_ref[...] / ncv
            gh = _relu(_dot(u_e, wg0e_ref[...]) + _dot(u_v, wg0v_ref[...])
                       + _dot(u_ref[...], wg0u_ref[...]) + bg0_ref[...])
            gh = _relu(_dot(gh, wg1_ref[...]) + bg1_ref[...])
            uo_ref[...] = _relu(_dot(gh, wg2_ref[...]) + bg2_ref[...])

    wspec = pl.BlockSpec((D, D), lambda i: (0, 0))
    bspec = pl.BlockSpec((1, D), lambda i: (0, 0))
    x_out, u_out = pl.pallas_call(
        node_body,
        grid=(nb,),
        in_specs=[
            pl.BlockSpec((NB, D), lambda i: (i, 0)),
            pl.BlockSpec((NB, D), lambda i: (i, 0)),
            pl.BlockSpec((NB, D), lambda i: (i, 0)),
            pl.BlockSpec((1, 1, NB), lambda i: (i, 0, 0)),
            pl.BlockSpec((B, D), lambda i: (0, 0)),
            wspec, wspec, wspec, wspec, wspec, bspec, bspec, bspec,
            wspec, wspec, wspec, wspec, wspec, bspec, bspec, bspec,
        ],
        out_specs=[pl.BlockSpec((NB, D), lambda i: (i, 0)),
                   pl.BlockSpec((B, D), lambda i: (0, 0))],
        out_shape=[jax.ShapeDtypeStruct((N, D), jnp.float32),
                   jax.ShapeDtypeStruct((B, D), jnp.float32)],
        scratch_shapes=[pltpu.VMEM((B, D), jnp.float32),
                        pltpu.VMEM((B, D), jnp.float32),
                        pltpu.VMEM((B, D), jnp.float32)],
    )(x, sums, ecnt, batch3, u,
      Wn0a, Wn0b, Wn0u, Wn1, Wn2,
      bn0.reshape(1, D), bn1.reshape(1, D), bn2.reshape(1, D),
      Wg0e, Wg0v, Wg0u, Wg1, Wg2,
      bg0.reshape(1, D), bg1.reshape(1, D), bg2.reshape(1, D))

    return (x_out, e_out, u_out)


# speedup vs baseline: 9.3286x; 1.0859x over previous
import functools

import jax
import jax.numpy as jnp
from jax import lax
from jax.experimental import pallas as pl
from jax.experimental.pallas import tpu as pltpu
from jax.experimental.pallas import tpu_sc as plsc

_NC = 2
_NS = 16


def _relu(v):
    return jnp.maximum(v, 0.0)


def _dot(a, b):
    return jnp.dot(a, b, preferred_element_type=jnp.float32)


def _seg_dot(a, b):
    return lax.dot_general(a, b, (((0,), (0,)), ((), ())),
                           preferred_element_type=jnp.float32)


def kernel(x, edge_index, edge_attr, u, batch,
           We0, be0, We1, be1, We2, be2,
           Wn0, bn0, Wn1, bn1, Wn2, bn2,
           Wg0, bg0, Wg1, bg1, Wg2, bg2):
    N, D = x.shape
    E = edge_attr.shape[0]
    B = u.shape[0]
    row, col = edge_index[0], edge_index[1]

    Wa, Wb, Wc, Wd = We0[:D], We0[D:2 * D], We0[2 * D:3 * D], We0[3 * D:]
    Wn0a, Wn0b, Wn0u = Wn0[:D], Wn0[D:2 * D], Wn0[2 * D:]
    Wg0e, Wg0v, Wg0u = Wg0[:D], Wg0[D:2 * D], Wg0[2 * D:]

    NB = 2000
    nb = N // NB
    batch3 = batch.reshape(nb, 1, NB)

    def onehot_t(b_row):
        return (b_row == lax.broadcasted_iota(jnp.int32, (B, NB), 0)
                ).astype(jnp.float32)

    def prep_body(x_ref, b_ref, u_ref, wa_ref, wb_ref, wd_ref, be0_ref,
                  xa_ref, xb_ref):
        xblk = x_ref[...]
        oh_t = onehot_t(b_ref[0])
        up = _dot(u_ref[...], wd_ref[...])
        xa_ref[...] = (_dot(xblk, wa_ref[...]) + _seg_dot(oh_t, up)
                       + be0_ref[...])
        xb_ref[...] = _dot(xblk, wb_ref[...])

    xa, xb = pl.pallas_call(
        prep_body,
        grid=(nb,),
        in_specs=[
            pl.BlockSpec((NB, D), lambda i: (i, 0)),
            pl.BlockSpec((1, 1, NB), lambda i: (i, 0, 0)),
            pl.BlockSpec((B, D), lambda i: (0, 0)),
            pl.BlockSpec((D, D), lambda i: (0, 0)),
            pl.BlockSpec((D, D), lambda i: (0, 0)),
            pl.BlockSpec((D, D), lambda i: (0, 0)),
            pl.BlockSpec((1, D), lambda i: (0, 0)),
        ],
        out_specs=[pl.BlockSpec((NB, D), lambda i: (i, 0)),
                   pl.BlockSpec((NB, D), lambda i: (i, 0))],
        out_shape=[jax.ShapeDtypeStruct((N, D), jnp.float32),
                   jax.ShapeDtypeStruct((N, D), jnp.float32)],
    )(x, batch3, u, Wa, Wb, Wd, be0.reshape(1, D))

    mesh = plsc.VectorSubcoreMesh(core_axis_name="c", subcore_axis_name="s",
                                  num_cores=_NC, num_subcores=_NS)
    CH = 200
    per_w = E // (_NC * _NS)
    NCHG = per_w // CH

    @functools.partial(
        pl.kernel,
        out_type=jax.ShapeDtypeStruct((E, D), jnp.float32),
        mesh=mesh,
        scratch_types=[
            [pltpu.VMEM((CH,), jnp.int32)] * 2,
            [pltpu.VMEM((CH,), jnp.int32)] * 2,
            [pltpu.VMEM((CH, D), jnp.float32)] * 2,
            [pltpu.VMEM((CH, D), jnp.float32)] * 2,
            [pltpu.SemaphoreType.DMA] * 2,
            [pltpu.SemaphoreType.DMA] * 2,
        ],
    )
    def gather_k(xa_h, xb_h, row_h, col_h, out_h,
                 idxr, idxc, bufa, bufb, sema, semb):
        wid = lax.axis_index("s") * _NC + lax.axis_index("c")
        base = wid * per_w

        def start(i, p):
            off = base + i * CH
            pltpu.sync_copy(row_h.at[pl.ds(off, CH)], idxr[p])
            pltpu.sync_copy(col_h.at[pl.ds(off, CH)], idxc[p])
            pltpu.async_copy(xa_h.at[idxr[p]], bufa[p], sema[p])
            pltpu.async_copy(xb_h.at[idxc[p]], bufb[p], semb[p])

        def finish(i, p):
            pltpu.make_async_copy(xa_h.at[idxr[p]], bufa[p], sema[p]).wait()
            pltpu.make_async_copy(xb_h.at[idxc[p]], bufb[p], semb[p]).wait()

            def addrow(j, c2):
                for k in range(D // 16):
                    sl = pl.ds(k * 16, 16)
                    bufa[p][j, sl] = bufa[p][j, sl] + bufb[p][j, sl]
                return c2

            lax.fori_loop(0, CH, addrow, 0)
            pltpu.sync_copy(bufa[p], out_h.at[pl.ds(base + i * CH, CH)])

        start(0, 0)

        def chunk(i, c):
            @pl.when(lax.rem(i, 2) == 0)
            def _():
                @pl.when(i + 1 < NCHG)
                def _():
                    start(i + 1, 1)
                finish(i, 0)

            @pl.when(lax.rem(i, 2) == 1)
            def _():
                @pl.when(i + 1 < NCHG)
                def _():
                    start(i + 1, 0)
                finish(i, 1)

            return c

        lax.fori_loop(0, NCHG, chunk, 0)

    g = gather_k(xa, xb, row, col)

    EB = 4000
    ne = E // EB

    def edge_body(g_ref, ea_ref, wc_ref, w1_ref, w2_ref, b1_ref, b2_ref,
                  out_ref):
        h = _relu(g_ref[...] + _dot(ea_ref[...], wc_ref[...]))
        h = _relu(_dot(h, w1_ref[...]) + b1_ref[...])
        out_ref[...] = _relu(_dot(h, w2_ref[...]) + b2_ref[...])

    e_out = pl.pallas_call(
        edge_body,
        grid=(ne,),
        in_specs=[
            pl.BlockSpec((EB, D), lambda i: (i, 0)),
            pl.BlockSpec((EB, D), lambda i: (i, 0)),
            pl.BlockSpec((D, D), lambda i: (0, 0)),
            pl.BlockSpec((D, D), lambda i: (0, 0)),
            pl.BlockSpec((D, D), lambda i: (0, 0)),
            pl.BlockSpec((1, D), lambda i: (0, 0)),
            pl.BlockSpec((1, D), lambda i: (0, 0)),
        ],
        out_specs=pl.BlockSpec((EB, D), lambda i: (i, 0)),
        out_shape=jax.ShapeDtypeStruct((E, D), jnp.float32),
    )(g, edge_attr, Wc, We1, We2, be1.reshape(1, D), be2.reshape(1, D))

    HN = N // _NC
    SPREAD = 512
    DBASE = (HN + 7) // 8 * 8
    per_tile = E // _NS
    WBS = (HN // _NS) // 8 * 8
    TS = HN - _NS * WBS
    WBC = (N // _NS) // 8 * 8
    TLC = N - _NS * WBC
    ZR = 104

    CHS = 160
    NCHS = per_tile // CHS

    def scatter_body(eout_h, row_h, sums_h, cnts_h,
                     idx, idx2, upd, ones1, zb2, zb1,
                     sem_l, sem_s, sem_c, sh_c, sh_s):
        cid = lax.axis_index("c")
        sid = lax.axis_index("s")

        def zrow(j, c):
            for k in range(D // 16):
                zb2[j, pl.ds(16 * k, 16)] = jnp.zeros((16,), jnp.float32)
            return c

        lax.fori_loop(0, ZR, zrow, 0)

        def fill1(buf, val, n16):
            def body(j, c):
                buf[pl.ds(j * 16, 16)] = jnp.full((16,), val, jnp.float32)
                return c
            lax.fori_loop(0, n16, body, 0)

        fill1(zb1, 0.0, WBC // 16)
        fill1(ones1, 1.0, CHS // 16)

        nfull = WBS // ZR
        rem = WBS - nfull * ZR

        def zinit(r, c):
            pltpu.sync_copy(zb2, sh_s.at[pl.ds(sid * WBS + r * ZR, ZR)])
            return c

        lax.fori_loop(0, nfull, zinit, 0)
        if rem:
            pltpu.sync_copy(zb2.at[pl.ds(0, rem)],
                            sh_s.at[pl.ds(sid * WBS + nfull * ZR, rem)])
        pltpu.sync_copy(zb1, sh_c.at[pl.ds(sid * WBC, WBC)])

        @pl.when(sid == _NS - 1)
        def _():
            if TS:
                pltpu.sync_copy(zb2.at[pl.ds(0, TS)],
                                sh_s.at[pl.ds(_NS * WBS, TS)])
            pltpu.sync_copy(zb1.at[pl.ds(0, TLC)],
                            sh_c.at[pl.ds(_NS * WBC, TLC)])

        plsc.subcore_barrier()

        base = sid * per_tile
        lo = cid * HN

        def load(i, p):
            off = base + i * CHS
            pltpu.sync_copy(row_h.at[pl.ds(off, CHS)], idx[p])
            pltpu.async_copy(eout_h.at[pl.ds(off, CHS)], upd[p], sem_l[p])

        def process(i, p):
            pltpu.make_async_copy(eout_h.at[pl.ds(base + i * CHS, CHS)],
                                  upd[p], sem_l[p]).wait()

            def remap(j, c2):
                iv = idx[p][pl.ds(j * 16, 16)]
                shf = iv - lo
                inb = (shf >= 0) & (shf < HN)
                idx2[p][pl.ds(j * 16, 16)] = jnp.where(
                    inb, shf, DBASE + (iv & (SPREAD - 1)))
                return c2

            lax.fori_loop(0, CHS // 16, remap, 0)

            @pl.when(cid == 0)
            def _():
                pltpu.async_copy(ones1, sh_c.at[idx[p]], sem_c[p], add=True)

            pltpu.async_copy(upd[p], sh_s.at[idx2[p]], sem_s[p], add=True)

        def wait_scatters(p):
            pltpu.make_async_copy(upd[p], sh_s.at[idx2[p]], sem_s[p]).wait()

            @pl.when(cid == 0)
            def _():
                pltpu.make_async_copy(ones1, sh_c.at[idx[p]],
                                      sem_c[p]).wait()

        load(0, 0)

        def chunk(i, c):
            @pl.when(lax.rem(i, 2) == 0)
            def _():
                @pl.when(i + 1 < NCHS)
                def _():
                    @pl.when(i >= 1)
                    def _():
                        wait_scatters(1)
                    load(i + 1, 1)
                process(i, 0)

            @pl.when(lax.rem(i, 2) == 1)
            def _():
                @pl.when(i + 1 < NCHS)
                def _():
                    wait_scatters(0)
                    load(i + 1, 0)
                process(i, 1)

            return c

        lax.fori_loop(0, NCHS, chunk, 0)
        wait_scatters((NCHS - 2) % 2)
        wait_scatters((NCHS - 1) % 2)
        plsc.subcore_barrier()

        def wback(r, c):
            rows = pl.ds(sid * WBS + r * ZR, ZR)
            pltpu.sync_copy(sh_s.at[rows], zb2)
            pltpu.sync_copy(
                zb2, sums_h.at[pl.ds(lo + sid * WBS + r * ZR, ZR)])
            return c

        lax.fori_loop(0, nfull, wback, 0)
        if rem:
            pltpu.sync_copy(sh_s.at[pl.ds(sid * WBS + nfull * ZR, rem)],
                            zb2.at[pl.ds(0, rem)])
            pltpu.sync_copy(
                zb2.at[pl.ds(0, rem)],
                sums_h.at[pl.ds(lo + sid * WBS + nfull * ZR, rem)])
        pltpu.sync_copy(sh_c.at[pl.ds(sid * WBC, WBC)], zb1)
        pltpu.sync_copy(zb1, cnts_h.at[pl.ds(cid * N + sid * WBC, WBC)])

        @pl.when(sid == _NS - 1)
        def _():
            if TS:
                tr = pl.ds(_NS * WBS, TS)
                pltpu.sync_copy(sh_s.at[tr], zb2.at[pl.ds(0, TS)])
                pltpu.sync_copy(zb2.at[pl.ds(0, TS)],
                                sums_h.at[pl.ds(lo + _NS * WBS, TS)])
            trc = pl.ds(_NS * WBC, TLC)
            pltpu.sync_copy(sh_c.at[trc], zb1.at[pl.ds(0, TLC)])
            pltpu.sync_copy(zb1.at[pl.ds(0, TLC)],
                            cnts_h.at[pl.ds(cid * N + _NS * WBC, TLC)])

    scatter_k = pl.kernel(
        scatter_body,
        out_type=(jax.ShapeDtypeStruct((N, D), jnp.float32),
                  jax.ShapeDtypeStruct((_NC * N,), jnp.float32)),
        mesh=mesh,
        scratch_types=[
            [pltpu.VMEM((CHS,), jnp.int32)] * 2,
            [pltpu.VMEM((CHS,), jnp.int32)] * 2,
            [pltpu.VMEM((CHS, D), jnp.float32)] * 2,
            pltpu.VMEM((CHS,), jnp.float32),
            pltpu.VMEM((ZR, D), jnp.float32),
            pltpu.VMEM((WBC,), jnp.float32),
            [pltpu.SemaphoreType.DMA] * 2,
            [pltpu.SemaphoreType.DMA] * 2,
            [pltpu.SemaphoreType.DMA] * 2,
            pltpu.VMEM_SHARED((N,), jnp.float32),
            pltpu.VMEM_SHARED((DBASE + SPREAD, D), jnp.float32),
        ],
    )

    sums, cnts = scatter_k(e_out, row)

    ecnt = jnp.broadcast_to(
        jnp.maximum(cnts[:N] + cnts[N:], 1.0)[:, None], (N, D))

    def node_body(x_ref, s_ref, ec_ref, b_ref, u_ref,
                  wn0a_ref, wn0b_ref, wn0u_ref, wn1_ref, wn2_ref,
                  bn0_ref, bn1_ref, bn2_ref,
                  wg0e_ref, wg0v_ref, wg0u_ref, wg1_ref, wg2_ref,
                  bg0_ref, bg1_ref, bg2_ref,
                  xo_ref, uo_ref, es_ref, vs_ref, nc_ref):
        i = pl.program_id(0)
        v_e = s_ref[...] / ec_ref[...]
        oh_t = onehot_t(b_ref[0])
        up = _dot(u_ref[...], wn0u_ref[...])
        h = _relu(_dot(x_ref[...], wn0a_ref[...]) + _dot(v_e, wn0b_ref[...])
                  + _seg_dot(oh_t, up) + bn0_ref[...])
        h = _relu(_dot(h, wn1_ref[...]) + bn1_ref[...])
        xo = _relu(_dot(h, wn2_ref[...]) + bn2_ref[...])
        xo_ref[...] = xo

        @pl.when(i == 0)
        def _():
            es_ref[...] = jnp.zeros((B, D), jnp.float32)
            vs_ref[...] = jnp.zeros((B, D), jnp.float32)
            nc_ref[...] = jnp.zeros((B, D), jnp.float32)

        es_ref[...] = es_ref[...] + _dot(oh_t, v_e)
        vs_ref[...] = vs_ref[...] + _dot(oh_t, xo)
        nc_ref[...] = nc_ref[...] + _dot(oh_t, jnp.ones((NB, D), jnp.float32))

        @pl.when(i == nb - 1)
        def _():
            ncv = jnp.maximum(nc_ref[...], 1.0)
            u_e = es_ref[...] / ncv
            u_v = vs_ref[...] / ncv
            gh = _relu(_dot(u_e, wg0e_ref[...]) + _dot(u_v, wg0v_ref[...])
                       + _dot(u_ref[...], wg0u_ref[...]) + bg0_ref[...])
            gh = _relu(_dot(gh, wg1_ref[...]) + bg1_ref[...])
            uo_ref[...] = _relu(_dot(gh, wg2_ref[...]) + bg2_ref[...])

    wspec = pl.BlockSpec((D, D), lambda i: (0, 0))
    bspec = pl.BlockSpec((1, D), lambda i: (0, 0))
    x_out, u_out = pl.pallas_call(
        node_body,
        grid=(nb,),
        in_specs=[
            pl.BlockSpec((NB, D), lambda i: (i, 0)),
            pl.BlockSpec((NB, D), lambda i: (i, 0)),
            pl.BlockSpec((NB, D), lambda i: (i, 0)),
            pl.BlockSpec((1, 1, NB), lambda i: (i, 0, 0)),
            pl.BlockSpec((B, D), lambda i: (0, 0)),
            wspec, wspec, wspec, wspec, wspec, bspec, bspec, bspec,
            wspec, wspec, wspec, wspec, wspec, bspec, bspec, bspec,
        ],
        out_specs=[pl.BlockSpec((NB, D), lambda i: (i, 0)),
                   pl.BlockSpec((B, D), lambda i: (0, 0))],
        out_shape=[jax.ShapeDtypeStruct((N, D), jnp.float32),
                   jax.ShapeDtypeStruct((B, D), jnp.float32)],
        scratch_shapes=[pltpu.VMEM((B, D), jnp.float32),
                        pltpu.VMEM((B, D), jnp.float32),
                        pltpu.VMEM((B, D), jnp.float32)],
    )(x, sums, ecnt, batch3, u,
      Wn0a, Wn0b, Wn0u, Wn1, Wn2,
      bn0.reshape(1, D), bn1.reshape(1, D), bn2.reshape(1, D),
      Wg0e, Wg0v, Wg0u, Wg1, Wg2,
      bg0.reshape(1, D), bg1.reshape(1, D), bg2.reshape(1, D))

    return (x_out, e_out, u_out)


# speedup vs baseline: 9.5184x; 1.0203x over previous
import functools

import jax
import jax.numpy as jnp
from jax import lax
from jax.experimental import pallas as pl
from jax.experimental.pallas import tpu as pltpu
from jax.experimental.pallas import tpu_sc as plsc

_NC = 2
_NS = 16


def _relu(v):
    return jnp.maximum(v, 0.0)


def _dot(a, b):
    return jnp.dot(a, b, preferred_element_type=jnp.float32)


def _seg_dot(a, b):
    return lax.dot_general(a, b, (((0,), (0,)), ((), ())),
                           preferred_element_type=jnp.float32)


def kernel(x, edge_index, edge_attr, u, batch,
           We0, be0, We1, be1, We2, be2,
           Wn0, bn0, Wn1, bn1, Wn2, bn2,
           Wg0, bg0, Wg1, bg1, Wg2, bg2):
    N, D = x.shape
    E = edge_attr.shape[0]
    B = u.shape[0]
    row, col = edge_index[0], edge_index[1]

    Wa, Wb, Wc, Wd = We0[:D], We0[D:2 * D], We0[2 * D:3 * D], We0[3 * D:]
    Wn0a, Wn0b, Wn0u = Wn0[:D], Wn0[D:2 * D], Wn0[2 * D:]
    Wg0e, Wg0v, Wg0u = Wg0[:D], Wg0[D:2 * D], Wg0[2 * D:]

    NB = 2000
    nb = N // NB
    batch3 = batch.reshape(nb, 1, NB)

    def onehot_t(b_row):
        return (b_row == lax.broadcasted_iota(jnp.int32, (B, NB), 0)
                ).astype(jnp.float32)

    def prep_body(x_ref, b_ref, u_ref, wa_ref, wb_ref, wd_ref, be0_ref,
                  xa_ref, xb_ref):
        xblk = x_ref[...]
        oh_t = onehot_t(b_ref[0])
        up = _dot(u_ref[...], wd_ref[...])
        xa_ref[...] = (_dot(xblk, wa_ref[...]) + _seg_dot(oh_t, up)
                       + be0_ref[...])
        xb_ref[...] = _dot(xblk, wb_ref[...])

    xa, xb = pl.pallas_call(
        prep_body,
        grid=(nb,),
        in_specs=[
            pl.BlockSpec((NB, D), lambda i: (i, 0)),
            pl.BlockSpec((1, 1, NB), lambda i: (i, 0, 0)),
            pl.BlockSpec((B, D), lambda i: (0, 0)),
            pl.BlockSpec((D, D), lambda i: (0, 0)),
            pl.BlockSpec((D, D), lambda i: (0, 0)),
            pl.BlockSpec((D, D), lambda i: (0, 0)),
            pl.BlockSpec((1, D), lambda i: (0, 0)),
        ],
        out_specs=[pl.BlockSpec((NB, D), lambda i: (i, 0)),
                   pl.BlockSpec((NB, D), lambda i: (i, 0))],
        out_shape=[jax.ShapeDtypeStruct((N, D), jnp.float32),
                   jax.ShapeDtypeStruct((N, D), jnp.float32)],
    )(x, batch3, u, Wa, Wb, Wd, be0.reshape(1, D))

    mesh = plsc.VectorSubcoreMesh(core_axis_name="c", subcore_axis_name="s",
                                  num_cores=_NC, num_subcores=_NS)
    CH = 200
    per_w = E // (_NC * _NS)
    NCHG = per_w // CH

    @functools.partial(
        pl.kernel,
        out_type=jax.ShapeDtypeStruct((E, D), jnp.float32),
        mesh=mesh,
        scratch_types=[
            [pltpu.VMEM((CH,), jnp.int32)] * 2,
            [pltpu.VMEM((CH,), jnp.int32)] * 2,
            [pltpu.VMEM((CH, D), jnp.float32)] * 2,
            [pltpu.VMEM((CH, D), jnp.float32)] * 2,
            [pltpu.SemaphoreType.DMA] * 2,
            [pltpu.SemaphoreType.DMA] * 2,
        ],
    )
    def gather_k(xa_h, xb_h, row_h, col_h, out_h,
                 idxr, idxc, bufa, bufb, sema, semb):
        wid = lax.axis_index("s") * _NC + lax.axis_index("c")
        base = wid * per_w

        def start(i, p):
            off = base + i * CH
            pltpu.sync_copy(row_h.at[pl.ds(off, CH)], idxr[p])
            pltpu.sync_copy(col_h.at[pl.ds(off, CH)], idxc[p])
            pltpu.async_copy(xa_h.at[idxr[p]], bufa[p], sema[p])
            pltpu.async_copy(xb_h.at[idxc[p]], bufb[p], semb[p])

        def finish(i, p):
            pltpu.make_async_copy(xa_h.at[idxr[p]], bufa[p], sema[p]).wait()
            pltpu.make_async_copy(xb_h.at[idxc[p]], bufb[p], semb[p]).wait()

            def addrow(j, c2):
                for k in range(D // 16):
                    sl = pl.ds(k * 16, 16)
                    bufa[p][j, sl] = bufa[p][j, sl] + bufb[p][j, sl]
                return c2

            lax.fori_loop(0, CH, addrow, 0)
            pltpu.sync_copy(bufa[p], out_h.at[pl.ds(base + i * CH, CH)])

        start(0, 0)

        def chunk(i, c):
            @pl.when(lax.rem(i, 2) == 0)
            def _():
                @pl.when(i + 1 < NCHG)
                def _():
                    start(i + 1, 1)
                finish(i, 0)

            @pl.when(lax.rem(i, 2) == 1)
            def _():
                @pl.when(i + 1 < NCHG)
                def _():
                    start(i + 1, 0)
                finish(i, 1)

            return c

        lax.fori_loop(0, NCHG, chunk, 0)

    g = gather_k(xa, xb, row, col)

    EB = 8000
    ne = E // EB

    def edge_body(g_ref, ea_ref, wc_ref, w1_ref, w2_ref, b1_ref, b2_ref,
                  out_ref):
        h = _relu(g_ref[...] + _dot(ea_ref[...], wc_ref[...]))
        h = _relu(_dot(h, w1_ref[...]) + b1_ref[...])
        out_ref[...] = _relu(_dot(h, w2_ref[...]) + b2_ref[...])

    e_out = pl.pallas_call(
        edge_body,
        grid=(ne,),
        in_specs=[
            pl.BlockSpec((EB, D), lambda i: (i, 0)),
            pl.BlockSpec((EB, D), lambda i: (i, 0)),
            pl.BlockSpec((D, D), lambda i: (0, 0)),
            pl.BlockSpec((D, D), lambda i: (0, 0)),
            pl.BlockSpec((D, D), lambda i: (0, 0)),
            pl.BlockSpec((1, D), lambda i: (0, 0)),
            pl.BlockSpec((1, D), lambda i: (0, 0)),
        ],
        out_specs=pl.BlockSpec((EB, D), lambda i: (i, 0)),
        out_shape=jax.ShapeDtypeStruct((E, D), jnp.float32),
    )(g, edge_attr, Wc, We1, We2, be1.reshape(1, D), be2.reshape(1, D))

    HN = N // _NC
    SPREAD = 512
    DBASE = (HN + 7) // 8 * 8
    per_tile = E // _NS
    WBS = (HN // _NS) // 8 * 8
    TS = HN - _NS * WBS
    WBC = (N // _NS) // 8 * 8
    TLC = N - _NS * WBC
    ZR = 104

    CHS = 160
    NCHS = per_tile // CHS

    def scatter_body(eout_h, row_h, sums_h, cnts_h,
                     idx, idx2, upd, ones1, zb2, zb1,
                     sem_l, sem_s, sem_c, sh_c, sh_s):
        cid = lax.axis_index("c")
        sid = lax.axis_index("s")

        def zrow(j, c):
            for k in range(D // 16):
                zb2[j, pl.ds(16 * k, 16)] = jnp.zeros((16,), jnp.float32)
            return c

        lax.fori_loop(0, ZR, zrow, 0)

        def fill1(buf, val, n16):
            def body(j, c):
                buf[pl.ds(j * 16, 16)] = jnp.full((16,), val, jnp.float32)
                return c
            lax.fori_loop(0, n16, body, 0)

        fill1(zb1, 0.0, WBC // 16)
        fill1(ones1, 1.0, CHS // 16)

        nfull = WBS // ZR
        rem = WBS - nfull * ZR

        def zinit(r, c):
            pltpu.sync_copy(zb2, sh_s.at[pl.ds(sid * WBS + r * ZR, ZR)])
            return c

        lax.fori_loop(0, nfull, zinit, 0)
        if rem:
            pltpu.sync_copy(zb2.at[pl.ds(0, rem)],
                            sh_s.at[pl.ds(sid * WBS + nfull * ZR, rem)])
        pltpu.sync_copy(zb1, sh_c.at[pl.ds(sid * WBC, WBC)])

        @pl.when(sid == _NS - 1)
        def _():
            if TS:
                pltpu.sync_copy(zb2.at[pl.ds(0, TS)],
                                sh_s.at[pl.ds(_NS * WBS, TS)])
            pltpu.sync_copy(zb1.at[pl.ds(0, TLC)],
                            sh_c.at[pl.ds(_NS * WBC, TLC)])

        plsc.subcore_barrier()

        base = sid * per_tile
        lo = cid * HN

        def load(i, p):
            off = base + i * CHS
            pltpu.sync_copy(row_h.at[pl.ds(off, CHS)], idx[p])
            pltpu.async_copy(eout_h.at[pl.ds(off, CHS)], upd[p], sem_l[p])

        def process(i, p):
            pltpu.make_async_copy(eout_h.at[pl.ds(base + i * CHS, CHS)],
                                  upd[p], sem_l[p]).wait()

            def remap(j, c2):
                iv = idx[p][pl.ds(j * 16, 16)]
                shf = iv - lo
                inb = (shf >= 0) & (shf < HN)
                idx2[p][pl.ds(j * 16, 16)] = jnp.where(
                    inb, shf, DBASE + (iv & (SPREAD - 1)))
                return c2

            lax.fori_loop(0, CHS // 16, remap, 0)

            @pl.when(cid == 0)
            def _():
                pltpu.async_copy(ones1, sh_c.at[idx[p]], sem_c[p], add=True)

            pltpu.async_copy(upd[p], sh_s.at[idx2[p]], sem_s[p], add=True)

        def wait_scatters(p):
            pltpu.make_async_copy(upd[p], sh_s.at[idx2[p]], sem_s[p]).wait()

            @pl.when(cid == 0)
            def _():
                pltpu.make_async_copy(ones1, sh_c.at[idx[p]],
                                      sem_c[p]).wait()

        load(0, 0)

        def chunk(i, c):
            @pl.when(lax.rem(i, 2) == 0)
            def _():
                @pl.when(i + 1 < NCHS)
                def _():
                    @pl.when(i >= 1)
                    def _():
                        wait_scatters(1)
                    load(i + 1, 1)
                process(i, 0)

            @pl.when(lax.rem(i, 2) == 1)
            def _():
                @pl.when(i + 1 < NCHS)
                def _():
                    wait_scatters(0)
                    load(i + 1, 0)
                process(i, 1)

            return c

        lax.fori_loop(0, NCHS, chunk, 0)
        wait_scatters((NCHS - 2) % 2)
        wait_scatters((NCHS - 1) % 2)
        plsc.subcore_barrier()

        def wback(r, c):
            rows = pl.ds(sid * WBS + r * ZR, ZR)
            pltpu.sync_copy(sh_s.at[rows], zb2)
            pltpu.sync_copy(
                zb2, sums_h.at[pl.ds(lo + sid * WBS + r * ZR, ZR)])
            return c

        lax.fori_loop(0, nfull, wback, 0)
        if rem:
            pltpu.sync_copy(sh_s.at[pl.ds(sid * WBS + nfull * ZR, rem)],
                            zb2.at[pl.ds(0, rem)])
            pltpu.sync_copy(
                zb2.at[pl.ds(0, rem)],
                sums_h.at[pl.ds(lo + sid * WBS + nfull * ZR, rem)])
        pltpu.sync_copy(sh_c.at[pl.ds(sid * WBC, WBC)], zb1)
        pltpu.sync_copy(zb1, cnts_h.at[pl.ds(cid * N + sid * WBC, WBC)])

        @pl.when(sid == _NS - 1)
        def _():
            if TS:
                tr = pl.ds(_NS * WBS, TS)
                pltpu.sync_copy(sh_s.at[tr], zb2.at[pl.ds(0, TS)])
                pltpu.sync_copy(zb2.at[pl.ds(0, TS)],
                                sums_h.at[pl.ds(lo + _NS * WBS, TS)])
            trc = pl.ds(_NS * WBC, TLC)
            pltpu.sync_copy(sh_c.at[trc], zb1.at[pl.ds(0, TLC)])
            pltpu.sync_copy(zb1.at[pl.ds(0, TLC)],
                            cnts_h.at[pl.ds(cid * N + _NS * WBC, TLC)])

    scatter_k = pl.kernel(
        scatter_body,
        out_type=(jax.ShapeDtypeStruct((N, D), jnp.float32),
                  jax.ShapeDtypeStruct((_NC * N,), jnp.float32)),
        mesh=mesh,
        scratch_types=[
            [pltpu.VMEM((CHS,), jnp.int32)] * 2,
            [pltpu.VMEM((CHS,), jnp.int32)] * 2,
            [pltpu.VMEM((CHS, D), jnp.float32)] * 2,
            pltpu.VMEM((CHS,), jnp.float32),
            pltpu.VMEM((ZR, D), jnp.float32),
            pltpu.VMEM((WBC,), jnp.float32),
            [pltpu.SemaphoreType.DMA] * 2,
            [pltpu.SemaphoreType.DMA] * 2,
            [pltpu.SemaphoreType.DMA] * 2,
            pltpu.VMEM_SHARED((N,), jnp.float32),
            pltpu.VMEM_SHARED((DBASE + SPREAD, D), jnp.float32),
        ],
    )

    sums, cnts = scatter_k(e_out, row)

    ecnt = jnp.broadcast_to(
        jnp.maximum(cnts[:N] + cnts[N:], 1.0)[:, None], (N, D))

    def node_body(x_ref, s_ref, ec_ref, b_ref, u_ref,
                  wn0a_ref, wn0b_ref, wn0u_ref, wn1_ref, wn2_ref,
                  bn0_ref, bn1_ref, bn2_ref,
                  wg0e_ref, wg0v_ref, wg0u_ref, wg1_ref, wg2_ref,
                  bg0_ref, bg1_ref, bg2_ref,
                  xo_ref, uo_ref, es_ref, vs_ref, nc_ref):
        i = pl.program_id(0)
        v_e = s_ref[...] / ec_ref[...]
        oh_t = onehot_t(b_ref[0])
        up = _dot(u_ref[...], wn0u_ref[...])
        h = _relu(_dot(x_ref[...], wn0a_ref[...]) + _dot(v_e, wn0b_ref[...])
                  + _seg_dot(oh_t, up) + bn0_ref[...])
        h = _relu(_dot(h, wn1_ref[...]) + bn1_ref[...])
        xo = _relu(_dot(h, wn2_ref[...]) + bn2_ref[...])
        xo_ref[...] = xo

        @pl.when(i == 0)
        def _():
            es_ref[...] = jnp.zeros((B, D), jnp.float32)
            vs_ref[...] = jnp.zeros((B, D), jnp.float32)
            nc_ref[...] = jnp.zeros((B, D), jnp.float32)

        es_ref[...] = es_ref[...] + _dot(oh_t, v_e)
        vs_ref[...] = vs_ref[...] + _dot(oh_t, xo)
        nc_ref[...] = nc_ref[...] + _dot(oh_t, jnp.ones((NB, D), jnp.float32))

        @pl.when(i == nb - 1)
        def _():
            ncv = jnp.maximum(nc_ref[...], 1.0)
            u_e = es_ref[...] / ncv
            u_v = vs_ref[...] / ncv
            gh = _relu(_dot(u_e, wg0e_ref[...]) + _dot(u_v, wg0v_ref[...])
                       + _dot(u_ref[...], wg0u_ref[...]) + bg0_ref[...])
            gh = _relu(_dot(gh, wg1_ref[...]) + bg1_ref[...])
            uo_ref[...] = _relu(_dot(gh, wg2_ref[...]) + bg2_ref[...])

    wspec = pl.BlockSpec((D, D), lambda i: (0, 0))
    bspec = pl.BlockSpec((1, D), lambda i: (0, 0))
    x_out, u_out = pl.pallas_call(
        node_body,
        grid=(nb,),
        in_specs=[
            pl.BlockSpec((NB, D), lambda i: (i, 0)),
            pl.BlockSpec((NB, D), lambda i: (i, 0)),
            pl.BlockSpec((NB, D), lambda i: (i, 0)),
            pl.BlockSpec((1, 1, NB), lambda i: (i, 0, 0)),
            pl.BlockSpec((B, D), lambda i: (0, 0)),
            wspec, wspec, wspec, wspec, wspec, bspec, bspec, bspec,
            wspec, wspec, wspec, wspec, wspec, bspec, bspec, bspec,
        ],
        out_specs=[pl.BlockSpec((NB, D), lambda i: (i, 0)),
                   pl.BlockSpec((B, D), lambda i: (0, 0))],
        out_shape=[jax.ShapeDtypeStruct((N, D), jnp.float32),
                   jax.ShapeDtypeStruct((B, D), jnp.float32)],
        scratch_shapes=[pltpu.VMEM((B, D), jnp.float32),
                        pltpu.VMEM((B, D), jnp.float32),
                        pltpu.VMEM((B, D), jnp.float32)],
    )(x, sums, ecnt, batch3, u,
      Wn0a, Wn0b, Wn0u, Wn1, Wn2,
      bn0.reshape(1, D), bn1.reshape(1, D), bn2.reshape(1, D),
      Wg0e, Wg0v, Wg0u, Wg1, Wg2,
      bg0.reshape(1, D), bg1.reshape(1, D), bg2.reshape(1, D))

    return (x_out, e_out, u_out)
